# baseline (speedup 1.0000x reference)
import functools

import jax
import jax.numpy as jnp
from jax import lax
from jax.experimental import pallas as pl
from jax.experimental.pallas import tpu as pltpu
from jax.experimental.pallas import tpu_sc as plsc

_K = 8192
_D = 256
_B = 8
_T = 1024
_N = _B * _T
_TM = 256
_COMMITMENT_COST = 0.25


def _vq_body(x_ref, w_ref, idx_ref, oh_ref, sc_ref,
             wn_ref, nw_ref, idxall_ref, acc_ref):
    i = pl.program_id(0)

    @pl.when(i == 0)
    def _():
        w = w_ref[...]
        nw = jnp.sqrt(jnp.sum(w * w, axis=-1, keepdims=True))
        nw_ref[...] = nw
        wn_ref[...] = w / jnp.clip(nw, 1e-12, None)
        acc_ref[0] = 0.0

    x = x_ref[...]
    s = jnp.sum(x * x, axis=-1, keepdims=True)
    n = jnp.sqrt(s)
    xn = x / jnp.clip(n, 1e-12, None)
    dist = lax.dot_general(
        xn, wn_ref[...], (((1,), (1,)), ((), ())),
        preferred_element_type=jnp.float32)
    kdim = dist.shape[1]
    m = jnp.max(dist, axis=1, keepdims=True)
    colf = lax.broadcasted_iota(jnp.int32, dist.shape, 1).astype(jnp.float32)
    vf = jnp.where(dist == m, colf, float(kdim))
    idxf = jnp.min(vf, axis=1, keepdims=True)
    oh = (vf == idxf).astype(jnp.float32)
    oh_ref[...] = oh
    idx = idxf.astype(jnp.int32)
    idx_ref[...] = idx.reshape(1, 1, idx.shape[0])

    w1 = lax.dot_general(oh, nw_ref[...], (((1,), (0,)), ((), ())),
                         preferred_element_type=jnp.float32)
    contrib = jnp.sum(w1 * w1 - 2.0 * (m * n) * w1 + s)
    acc_ref[0] += contrib

    tiles_per_b = _T // _TM
    b = i // tiles_per_b
    tcol = (i % tiles_per_b) * _TM
    idxall_ref[pl.ds(b, 1), pl.ds(tcol, _TM)] = idx.reshape(1, _TM)

    @pl.when(i == pl.num_programs(0) - 1)
    def _():
        ia = idxall_ref[...]
        eq = (ia[:, None, :] == ia[None, :, :]).astype(jnp.float32)
        c = jnp.sum(eq, axis=0)
        ent = jnp.sum(jnp.log(c * (1.0 / _B) + 1e-10)) * (1.0 / _B)
        msum = acc_ref[0] * (1.0 / (_N * _D))
        sc_ref[0] = msum
        sc_ref[1] = _COMMITMENT_COST * msum
        sc_ref[2] = jnp.exp(-ent)


def _vq_main(x_flat, w):
    n, d = x_flat.shape
    k = w.shape[0]
    g = n // _TM
    idx3, onehot, scalars = pl.pallas_call(
        _vq_body,
        grid=(g,),
        in_specs=[
            pl.BlockSpec((_TM, d), lambda i: (i, 0)),
            pl.BlockSpec((k, d), lambda i: (0, 0)),
        ],
        out_specs=[
            pl.BlockSpec((1, 1, _TM), lambda i: (i, 0, 0)),
            pl.BlockSpec((_TM, k), lambda i: (i, 0)),
            pl.BlockSpec(memory_space=pltpu.MemorySpace.SMEM),
        ],
        out_shape=[
            jax.ShapeDtypeStruct((g, 1, _TM), jnp.int32),
            jax.ShapeDtypeStruct((n, k), jnp.float32),
            jax.ShapeDtypeStruct((4,), jnp.float32),
        ],
        scratch_shapes=[
            pltpu.VMEM((k, d), jnp.float32),
            pltpu.VMEM((k, 1), jnp.float32),
            pltpu.VMEM((_B, _T), jnp.int32),
            pltpu.SMEM((1,), jnp.float32),
        ],
    )(x_flat, w)
    return idx3.reshape(n), onehot, scalars


@functools.lru_cache(maxsize=None)
def _make_sc_gather(n, d):
    info = plsc.get_sparse_core_info()
    nw = info.num_cores * info.num_subcores
    bpw = n // nw
    mesh = plsc.VectorSubcoreMesh(core_axis_name="c", subcore_axis_name="s")

    @functools.partial(
        pl.kernel,
        out_type=jax.ShapeDtypeStruct((n, d), jnp.float32),
        mesh=mesh,
        scratch_types=[
            pltpu.VMEM((bpw,), jnp.int32),
            pltpu.VMEM((bpw, d), jnp.float32),
            pltpu.SemaphoreType.DMA,
        ],
    )
    def gather_kernel(table_hbm, idx_hbm, out_hbm, idx_v, rows_v, sem):
        wid = lax.axis_index("s") * info.num_cores + lax.axis_index("c")
        base = wid * bpw
        pltpu.sync_copy(idx_hbm.at[pl.ds(base, bpw)], idx_v)
        pltpu.async_copy(table_hbm.at[idx_v], rows_v, sem).wait()
        pltpu.sync_copy(rows_v, out_hbm.at[pl.ds(base, bpw)])

    return gather_kernel


def kernel(inputs, weight):
    b, t, d = inputs.shape
    k = weight.shape[0]
    x_flat = inputs.reshape(b * t, d)
    idx_flat, onehot, scalars = _vq_main(x_flat, weight)
    quant = _make_sc_gather(b * t, d)(weight, idx_flat)
    quantized_st = quant.reshape(b, t, d)
    encoding_indices_out = idx_flat.reshape(b, t, 1)
    min_encodings = onehot.reshape(b, t, k)
    return (quantized_st, encoding_indices_out, scalars[0], scalars[1],
            scalars[2], min_encodings)
